# Initial kernel scaffold; baseline (speedup 1.0000x reference)
import jax
import jax.numpy as jnp
from jax.experimental import pallas as pl


def kernel(x, emb_tables, W, b, gamma, beta):
    raise NotImplementedError("write your pallas kernel here")


# speedup vs baseline: 2.0587x; 2.0587x over previous
import functools

import jax
import jax.numpy as jnp
from jax import lax
from jax.experimental import pallas as pl
from jax.experimental.pallas import tpu as pltpu
from jax.experimental.pallas import tpu_sc as plsc

NV = 26
DEPTH = 32
DE = 7
DM = 128
B = 1024
NTOK = NV * B * DEPTH
NROWS = 3 * NV

_NC = 2
_NS = 16
_NW = _NC * _NS
_RPW = NV * B // _NW
_RS = 4
_S = _RS * DEPTH
_STEPS = _RPW // _RS


def _lut_body(emb_ref, w_ref, b_ref, g_ref, bt_ref, lut_ref):
    rowmask = (lax.broadcasted_iota(jnp.int32, (3, 1), 0) < 2).astype(jnp.float32)
    e = emb_ref[0] * rowmask
    h = lax.dot_general(e, w_ref[0], (((1,), (0,)), ((), ())),
                        preferred_element_type=jnp.float32)
    h = h + b_ref[0]
    mu = jnp.mean(h, axis=-1, keepdims=True)
    var = jnp.mean((h - mu) ** 2, axis=-1, keepdims=True)
    lut_ref[0] = (h - mu) * lax.rsqrt(var + 1e-5) * g_ref[0] + bt_ref[0]


def _lut(emb_tables, W, b3, g3, bt3):
    return pl.pallas_call(
        _lut_body,
        grid=(NV,),
        in_specs=[
            pl.BlockSpec((1, 3, DE), lambda i: (i, 0, 0)),
            pl.BlockSpec((1, DE, DM), lambda i: (i, 0, 0)),
            pl.BlockSpec((1, 1, DM), lambda i: (i, 0, 0)),
            pl.BlockSpec((1, 1, DM), lambda i: (i, 0, 0)),
            pl.BlockSpec((1, 1, DM), lambda i: (i, 0, 0)),
        ],
        out_specs=pl.BlockSpec((1, 3, DM), lambda i: (i, 0, 0)),
        out_shape=jax.ShapeDtypeStruct((NV, 3, DM), jnp.float32),
    )(emb_tables, W, b3, g3, bt3)


def _sc_body(lut_hbm, x_hbm, out_hbm, x_v, idx_v, rows_v, sem_g):
    wid = lax.axis_index("s") * _NC + lax.axis_index("c")
    base_r = wid * _RPW

    def step(s, carry):
        r0 = base_r + s * _RS
        i = r0 // B
        b0 = r0 % B
        pltpu.sync_copy(x_hbm.at[pl.ds(b0, _RS), pl.ds(i, 1)], x_v)
        for g in range(_S // 16):
            xv = x_v[g // 2, 0, pl.ds((g % 2) * 16, 16)]
            c = jnp.where(xv < 255, jnp.minimum(xv, 1), 2)
            idx_v[pl.ds(g * 16, 16)] = 3 * i + c
        pltpu.async_copy(lut_hbm.at[idx_v], rows_v, sem_g).wait()
        pltpu.sync_copy(rows_v, out_hbm.at[pl.ds(r0 * DEPTH, _S)])
        return carry

    lax.fori_loop(0, _STEPS, step, 0)


@functools.cache
def _sc_expand():
    return functools.partial(
        pl.kernel,
        out_type=jax.ShapeDtypeStruct((NTOK, DM), jnp.float32),
        mesh=plsc.VectorSubcoreMesh(core_axis_name="c", subcore_axis_name="s"),
        scratch_types=[
            pltpu.VMEM((_RS, 1, DEPTH), jnp.int32),
            pltpu.VMEM((_S,), jnp.int32),
            pltpu.VMEM((_S, DM), jnp.float32),
            pltpu.SemaphoreType.DMA,
        ],
    )(_sc_body)


def kernel(x, emb_tables, W, b, gamma, beta):
    lut = _lut(emb_tables, W,
               b.reshape(NV, 1, DM),
               gamma.reshape(NV, 1, DM),
               beta.reshape(NV, 1, DM))
    out = _sc_expand()(lut.reshape(NROWS, DM), x.astype(jnp.int32))
    return out.reshape(NV, B, DEPTH, DM)


# speedup vs baseline: 13.6384x; 6.6249x over previous
import functools

import jax
import jax.numpy as jnp
from jax import lax
from jax.experimental import pallas as pl
from jax.experimental.pallas import tpu as pltpu
from jax.experimental.pallas import tpu_sc as plsc

NV = 26
DEPTH = 32
DE = 7
DM = 128
B = 1024
NTOK = NV * B * DEPTH
_LR = 8
NROWS = _LR * NV

_NC = 2
_NS = 16
_NW = _NC * _NS
_RPW = NV * B // _NW
_RS = 4
_S = _RS * DEPTH
_STEPS = _RPW // _RS
_NB = 4


def _lut_body(emb_ref, w_ref, b_ref, g_ref, bt_ref, lut_ref):
    rowmask = (lax.broadcasted_iota(jnp.int32, (_LR, 1), 0) < 2).astype(jnp.float32)
    e = jnp.concatenate(
        [emb_ref[0], jnp.zeros((_LR - 3, DE), jnp.float32)]) * rowmask
    h = lax.dot_general(e, w_ref[0], (((1,), (0,)), ((), ())),
                        preferred_element_type=jnp.float32)
    h = h + b_ref[0]
    mu = jnp.mean(h, axis=-1, keepdims=True)
    var = jnp.mean((h - mu) ** 2, axis=-1, keepdims=True)
    lut_ref[0] = (h - mu) * lax.rsqrt(var + 1e-5) * g_ref[0] + bt_ref[0]


def _lut(emb_tables, W, b3, g3, bt3):
    return pl.pallas_call(
        _lut_body,
        grid=(NV,),
        in_specs=[
            pl.BlockSpec((1, 3, DE), lambda i: (i, 0, 0)),
            pl.BlockSpec((1, DE, DM), lambda i: (i, 0, 0)),
            pl.BlockSpec((1, 1, DM), lambda i: (i, 0, 0)),
            pl.BlockSpec((1, 1, DM), lambda i: (i, 0, 0)),
            pl.BlockSpec((1, 1, DM), lambda i: (i, 0, 0)),
        ],
        out_specs=pl.BlockSpec((1, _LR, DM), lambda i: (i, 0, 0)),
        out_shape=jax.ShapeDtypeStruct((NV, _LR, DM), jnp.float32),
    )(emb_tables, W, b3, g3, bt3)


def _sc_body(lut_hbm, x_hbm, out_hbm, lut_s,
             x_v, idx_v, rows_v, sem_x, sem_g, sem_o):
    sid = lax.axis_index("s")
    wid = sid * _NC + lax.axis_index("c")
    base_r = wid * _RPW

    @pl.when(sid == 0)
    def _():
        pltpu.sync_copy(lut_hbm, lut_s)
    plsc.subcore_barrier()

    def fire_x(s, j):
        r0 = base_r + s * _RS
        pltpu.async_copy(x_hbm.at[pl.ds(r0 % B, _RS), pl.ds(r0 // B, 1)],
                         x_v[j], sem_x[j])

    def wait_x(j):
        pltpu.make_async_copy(x_hbm.at[pl.ds(0, _RS), pl.ds(0, 1)],
                              x_v[j], sem_x[j]).wait()

    def wait_o(j):
        pltpu.make_async_copy(rows_v[j], out_hbm.at[pl.ds(0, _S)],
                              sem_o[j]).wait()

    for j in range(_NB):
        fire_x(j, j)

    def step(k, carry):
        descs = []
        for j in range(_NB):
            s = k * _NB + j
            r0 = base_r + s * _RS
            rowb = _LR * (r0 // B)
            wait_x(j)

            @pl.when(k >= 1)
            def _():
                wait_o(j)

            for g in range(_S // 16):
                xv = x_v[j][g // 2, 0, pl.ds((g % 2) * 16, 16)]
                c = jnp.where(xv < 255, jnp.minimum(xv, 1), 2)
                idx_v[j][pl.ds(g * 16, 16)] = rowb + c

            descs.append(pltpu.async_copy(lut_s.at[idx_v[j]], rows_v[j],
                                          sem_g[j]))

            @pl.when(k < (_STEPS // _NB) - 1)
            def _():
                fire_x(s + _NB, j)

        for j in range(_NB):
            s = k * _NB + j
            r0 = base_r + s * _RS
            descs[j].wait()
            pltpu.async_copy(rows_v[j], out_hbm.at[pl.ds(r0 * DEPTH, _S)],
                             sem_o[j])
        return carry

    lax.fori_loop(0, _STEPS // _NB, step, 0)
    for j in range(_NB):
        wait_o(j)


@functools.cache
def _sc_expand():
    return functools.partial(
        pl.kernel,
        out_type=jax.ShapeDtypeStruct((NTOK, DM), jnp.float32),
        mesh=plsc.VectorSubcoreMesh(core_axis_name="c", subcore_axis_name="s"),
        scratch_types=[
            pltpu.VMEM_SHARED((NROWS, DM), jnp.float32),
            [pltpu.VMEM((_RS, 1, DEPTH), jnp.int32) for _ in range(_NB)],
            [pltpu.VMEM((_S,), jnp.int32) for _ in range(_NB)],
            [pltpu.VMEM((_S, DM), jnp.float32) for _ in range(_NB)],
            [pltpu.SemaphoreType.DMA for _ in range(_NB)],
            [pltpu.SemaphoreType.DMA for _ in range(_NB)],
            [pltpu.SemaphoreType.DMA for _ in range(_NB)],
        ],
    )(_sc_body)


def kernel(x, emb_tables, W, b, gamma, beta):
    lut = _lut(emb_tables, W,
               b.reshape(NV, 1, DM),
               gamma.reshape(NV, 1, DM),
               beta.reshape(NV, 1, DM))
    out = _sc_expand()(lut.reshape(NROWS, DM), x.astype(jnp.int32))
    return out.reshape(NV, B, DEPTH, DM)


# speedup vs baseline: 15.1103x; 1.1079x over previous
import functools

import jax
import jax.numpy as jnp
from jax import lax
from jax.experimental import pallas as pl
from jax.experimental.pallas import tpu as pltpu
from jax.experimental.pallas import tpu_sc as plsc

NV = 26
DEPTH = 32
DE = 7
DM = 128
B = 1024
NTOK = NV * B * DEPTH
_LR = 8
NROWS = _LR * NV

_NC = 2
_NS = 16
_NW = _NC * _NS
_RPW = NV * B // _NW
_RS = 4
_S = _RS * DEPTH
_STEPS = _RPW // _RS
_NB = 4


def _lut_body(emb_ref, w_ref, b_ref, g_ref, bt_ref, lut_ref):
    rowmask = (lax.broadcasted_iota(jnp.int32, (_LR, 1), 0) < 2).astype(jnp.float32)
    e = jnp.concatenate(
        [emb_ref[0], jnp.zeros((_LR - 3, DE), jnp.float32)]) * rowmask
    h = lax.dot_general(e, w_ref[0], (((1,), (0,)), ((), ())),
                        preferred_element_type=jnp.float32)
    h = h + b_ref[0]
    mu = jnp.mean(h, axis=-1, keepdims=True)
    var = jnp.mean((h - mu) ** 2, axis=-1, keepdims=True)
    lut_ref[0] = (h - mu) * lax.rsqrt(var + 1e-5) * g_ref[0] + bt_ref[0]


def _lut(emb_tables, W, b3, g3, bt3):
    return pl.pallas_call(
        _lut_body,
        grid=(NV,),
        in_specs=[
            pl.BlockSpec((1, 3, DE), lambda i: (i, 0, 0)),
            pl.BlockSpec((1, DE, DM), lambda i: (i, 0, 0)),
            pl.BlockSpec((1, 1, DM), lambda i: (i, 0, 0)),
            pl.BlockSpec((1, 1, DM), lambda i: (i, 0, 0)),
            pl.BlockSpec((1, 1, DM), lambda i: (i, 0, 0)),
        ],
        out_specs=pl.BlockSpec((1, _LR, DM), lambda i: (i, 0, 0)),
        out_shape=jax.ShapeDtypeStruct((NV, _LR, DM), jnp.float32),
    )(emb_tables, W, b3, g3, bt3)


def _sc_body(lut_hbm, x_hbm, out_hbm, lut_s,
             x_v, idx_v, rows_v, sem_x, sem_g, sem_o):
    sid = lax.axis_index("s")
    wid = sid * _NC + lax.axis_index("c")
    base_r = wid * _RPW

    @pl.when(sid == 0)
    def _():
        pltpu.sync_copy(lut_hbm, lut_s)
    plsc.subcore_barrier()

    def fire_x(s, j):
        r0 = base_r + s * _RS
        pltpu.async_copy(x_hbm.at[pl.ds(r0 % B, _RS), pl.ds(r0 // B, 1)],
                         x_v[j], sem_x[j])

    def wait_x(j):
        pltpu.make_async_copy(x_hbm.at[pl.ds(0, _RS), pl.ds(0, 1)],
                              x_v[j], sem_x[j]).wait()

    def wait_o(j):
        pltpu.make_async_copy(rows_v[j], out_hbm.at[pl.ds(0, _S)],
                              sem_o[j]).wait()

    for j in range(_NB):
        fire_x(j, j)

    def step(k, carry):
        descs = []
        for j in range(_NB):
            s = k * _NB + j
            r0 = base_r + s * _RS
            rowb = _LR * (r0 // B)
            wait_x(j)

            @pl.when(k >= 1)
            def _():
                wait_o(j)

            for g in range(_S // 16):
                xv = x_v[j][g // 2, 0, pl.ds((g % 2) * 16, 16)]
                c = jnp.where(xv < 255, jnp.minimum(xv, 1), 2)
                idx_v[j][pl.ds(g * 16, 16)] = rowb + c

            descs.append(pltpu.async_copy(lut_s.at[idx_v[j]], rows_v[j],
                                          sem_g[j]))

            @pl.when(k < (_STEPS // _NB) - 1)
            def _():
                fire_x(s + _NB, j)

        for j in range(_NB):
            s = k * _NB + j
            r0 = base_r + s * _RS
            descs[j].wait()
            pltpu.async_copy(rows_v[j], out_hbm.at[pl.ds(r0 * DEPTH, _S)],
                             sem_o[j])
        return carry

    lax.fori_loop(0, _STEPS // _NB, step, 0)
    for j in range(_NB):
        wait_o(j)


@functools.cache
def _sc_expand():
    return functools.partial(
        pl.kernel,
        out_type=jax.ShapeDtypeStruct((NTOK, DM), jnp.float32),
        mesh=plsc.VectorSubcoreMesh(core_axis_name="c", subcore_axis_name="s"),
        scratch_types=[
            pltpu.VMEM_SHARED((NROWS, DM), jnp.float32),
            [pltpu.VMEM((_RS, 1, DEPTH), jnp.int32) for _ in range(_NB)],
            [pltpu.VMEM((_S,), jnp.int32) for _ in range(_NB)],
            [pltpu.VMEM((_S, DM), jnp.float32) for _ in range(_NB)],
            [pltpu.SemaphoreType.DMA for _ in range(_NB)],
            [pltpu.SemaphoreType.DMA for _ in range(_NB)],
            [pltpu.SemaphoreType.DMA for _ in range(_NB)],
        ],
    )(_sc_body)


_BBLK = 16


def _tc_expand_body(x_ref, lut_ref, out_ref):
    for i in range(NV):
        xi = x_ref[:, i, :]
        f0 = (xi == 0).astype(jnp.float32)[..., None]
        f1 = ((xi >= 1) & (xi < 255)).astype(jnp.float32)[..., None]
        r0 = lut_ref[i, 0]
        r1 = lut_ref[i, 1]
        r2 = lut_ref[i, 2]
        out_ref[i] = r2 + f0 * (r0 - r2) + f1 * (r1 - r2)


def _tc_expand(x3, lut):
    return pl.pallas_call(
        _tc_expand_body,
        grid=(B // _BBLK,),
        in_specs=[
            pl.BlockSpec((_BBLK, NV, DEPTH), lambda g: (g, 0, 0)),
            pl.BlockSpec((NV, _LR, DM), lambda g: (0, 0, 0)),
        ],
        out_specs=pl.BlockSpec((NV, _BBLK, DEPTH, DM), lambda g: (0, g, 0, 0)),
        out_shape=jax.ShapeDtypeStruct((NV, B, DEPTH, DM), jnp.float32),
    )(x3, lut)


def kernel(x, emb_tables, W, b, gamma, beta):
    lut = _lut(emb_tables, W,
               b.reshape(NV, 1, DM),
               gamma.reshape(NV, 1, DM),
               beta.reshape(NV, 1, DM))
    return _tc_expand(x.astype(jnp.int32), lut)
